# Initial kernel scaffold; baseline (speedup 1.0000x reference)
import jax
import jax.numpy as jnp
from jax.experimental import pallas as pl


def kernel(feat_src, feat_dst, edge_index, dst_node_transformation_weight, src_node_transformation_weight, relation_embedding, relation_transformation_weight):
    raise NotImplementedError("write your pallas kernel here")


# speedup vs baseline: 3.4939x; 3.4939x over previous
import functools
import jax
import jax.numpy as jnp
from jax import lax
from jax.experimental import pallas as pl
from jax.experimental.pallas import tpu as pltpu
from jax.experimental.pallas import tpu_sc as plsc

_N = 10000
_E = 160000
_D = 256
_H = 8
_F = 64
_HF = _H * _F
_NB = 10
_BN = _N // _NB
_C = 128
_NCHUNK = _E // _C
_N2 = 10240
_R = 1024
_RT = _R // 16


def _dense_body(fsrc, fdst, wsrc, wdst, rele, wrel, fs_o, es_o, ed_o):
    f32 = jnp.float32
    rel_att = lax.dot_general(rele[...], wrel[...], (((1,), (0,)), ((), ())),
                              preferred_element_type=f32)
    qd = jnp.concatenate([rel_att[:, h * 2 * _F: h * 2 * _F + _F]
                          for h in range(_H)], axis=1)
    qs = jnp.concatenate([rel_att[:, h * 2 * _F + _F: (h + 1) * 2 * _F]
                          for h in range(_H)], axis=1)
    j64 = lax.broadcasted_iota(jnp.int32, (_HF, 16), 0) // _F
    l8 = lax.broadcasted_iota(jnp.int32, (_HF, 16), 1) % _H
    sel = jnp.where(j64 == l8, f32(1.0), f32(0.0))
    ws8 = lax.dot_general(wsrc[...] * qs, sel, (((1,), (0,)), ((), ())),
                          preferred_element_type=f32)
    wd8 = lax.dot_general(wdst[...] * qd, sel, (((1,), (0,)), ((), ())),
                          preferred_element_type=f32)
    fs_o[...] = lax.dot_general(fsrc[...], wsrc[...], (((1,), (0,)), ((), ())),
                                preferred_element_type=f32)
    es_o[...] = lax.dot_general(fsrc[...], ws8, (((1,), (0,)), ((), ())),
                                preferred_element_type=f32)
    ed_o[...] = lax.dot_general(fdst[...], wd8, (((1,), (0,)), ((), ())),
                                preferred_element_type=f32)


def _dense(feat_src, feat_dst, wsrc, wdst, rele, wrel):
    full = lambda shape: pl.BlockSpec(shape, lambda i: (0, 0))
    row = lambda w: pl.BlockSpec((_BN, w), lambda i: (i, 0))
    return pl.pallas_call(
        _dense_body,
        grid=(_NB,),
        in_specs=[row(_D), row(_D), full((_D, _HF)), full((_D, _HF)),
                  full((1, _D)), full((_D, 2 * _HF))],
        out_specs=[row(_HF), row(16), row(16)],
        out_shape=[jax.ShapeDtypeStruct((_N, _HF), jnp.float32),
                   jax.ShapeDtypeStruct((_N, 16), jnp.float32),
                   jax.ShapeDtypeStruct((_N, 16), jnp.float32)],
    )(feat_src, feat_dst, wsrc, wdst, rele, wrel)


def _attn_body(es_hbm, ed_hbm, src_hbm, dst_hbm, ex_hbm, denp_hbm,
               idx_s, idx_d, es_b, ed_b, ex_b, zb, den_sh, sem):
    cid = lax.axis_index("c")
    sid = lax.axis_index("s")
    w = cid * 16 + sid

    def zrow(i, c):
        zb[i, :] = jnp.zeros((16,), jnp.float32)
        return c
    lax.fori_loop(0, _N2 // 16, zrow, 0)
    pltpu.sync_copy(zb, den_sh.at[pl.ds(sid * (_N2 // 16), _N2 // 16)])
    plsc.subcore_barrier()

    nchunks = jnp.where(w < _NCHUNK - 32 * (_NCHUNK // 32),
                        _NCHUNK // 32 + 1, _NCHUNK // 32)

    def chunk(k, c):
        base = (w + 32 * k) * _C
        pltpu.sync_copy(src_hbm.at[pl.ds(base, _C)], idx_s)
        pltpu.sync_copy(dst_hbm.at[pl.ds(base, _C)], idx_d)
        pltpu.async_copy(es_hbm.at[idx_s], es_b, sem).wait()
        pltpu.async_copy(ed_hbm.at[idx_d], ed_b, sem).wait()

        z16 = jnp.zeros((16,), jnp.float32)
        s16 = jnp.full((16,), 0.2, jnp.float32)

        def per_edge(i, cc):
            v = es_b[i, :] + ed_b[i, :]
            v = jnp.maximum(v, z16) + s16 * jnp.minimum(v, z16)
            ex_b[i, :] = jnp.exp(v)
            return cc
        lax.fori_loop(0, _C, per_edge, 0)
        pltpu.sync_copy(ex_b, ex_hbm.at[pl.ds(base, _C)])
        pltpu.sync_copy(ex_b, den_sh.at[idx_d], add=True)
        return c
    lax.fori_loop(0, nchunks, chunk, 0)
    plsc.subcore_barrier()
    rows = _N2 // 16
    pltpu.sync_copy(den_sh.at[pl.ds(sid * rows, rows)],
                    denp_hbm.at[pl.ds(cid * _N2 + sid * rows, rows)])


def _attn(es, ed, src, dst):
    mesh = plsc.VectorSubcoreMesh(core_axis_name="c", subcore_axis_name="s")
    k = pl.kernel(
        _attn_body,
        compiler_params=pltpu.CompilerParams(use_tc_tiling_on_sc=False),
        out_type=[jax.ShapeDtypeStruct((_E, 16), jnp.float32),
                  jax.ShapeDtypeStruct((2 * _N2, 16), jnp.float32)],
        mesh=mesh,
        scratch_types=[
            pltpu.VMEM((_C,), jnp.int32),
            pltpu.VMEM((_C,), jnp.int32),
            pltpu.VMEM((_C, 16), jnp.float32),
            pltpu.VMEM((_C, 16), jnp.float32),
            pltpu.VMEM((_C, 16), jnp.float32),
            pltpu.VMEM((_N2 // 16, 16), jnp.float32),
            pltpu.VMEM_SHARED((_N2, 16), jnp.float32),
            pltpu.SemaphoreType.DMA,
        ],
    )
    return k(es, ed, src, dst)


def _msg_body(fs_hbm, ex_hbm, denp_hbm, src_hbm, dst_hbm, outp_hbm,
              idx_s, idx_d, idx_d2, lidx, ex_b, d0, d1, a_b, fs_b, zb,
              acc, sem):
    cid = lax.axis_index("c")
    sid = lax.axis_index("s")

    def zrow(i, c):
        for j in range(_HF // 16):
            zb[i, pl.ds(j * 16, 16)] = jnp.zeros((16,), jnp.float32)
        return c
    lax.fori_loop(0, 40, zrow, 0)

    nchunks = jnp.where(sid < _NCHUNK - 16 * (_NCHUNK // 16),
                        _NCHUNK // 16 + 1, _NCHUNK // 16)

    def one_pass(p, carry):
        lo = pl.multiple_of(cid * (5 * _R) + p * _R, _R)
        r0 = sid * _RT
        for z in range(2):
            pltpu.sync_copy(zb.at[pl.ds(0, 32)], acc.at[pl.ds(r0 + 32 * z, 32)])
        plsc.subcore_barrier()

        def chunk(k, c):
            base = (sid + 16 * k) * _C
            pltpu.sync_copy(src_hbm.at[pl.ds(base, _C)], idx_s)
            pltpu.sync_copy(dst_hbm.at[pl.ds(base, _C)], idx_d)
            pltpu.async_copy(fs_hbm.at[idx_s], fs_b, sem).wait()
            pltpu.sync_copy(ex_hbm.at[pl.ds(base, _C)], ex_b)
            pltpu.async_copy(denp_hbm.at[idx_d], d0, sem).wait()
            for j in range(_C // 16):
                idx_d2[pl.ds(j * 16, 16)] = (idx_d[pl.ds(j * 16, 16)]
                                             + jnp.int32(_N2))
            pltpu.async_copy(denp_hbm.at[idx_d2], d1, sem).wait()

            lov = jnp.full((16,), lo, jnp.int32)
            hiv = jnp.full((16,), lo + _R, jnp.int32)
            z16i = jnp.zeros((16,), jnp.int32)
            z16f = jnp.zeros((16,), jnp.float32)
            one16 = jnp.ones((16,), jnp.float32)
            eps16 = jnp.full((16,), 1e-16, jnp.float32)

            def per_group(g, cc):
                dvec = idx_d[pl.ds(g * 16, 16)]
                inb = (dvec >= lov) & (dvec < hiv)
                lidx[pl.ds(g * 16, 16)] = jnp.where(inb, dvec - lov, z16i)
                mf = jnp.where(inb, one16, z16f)
                for lane in range(16):
                    i = g * 16 + lane
                    den = d0[i, :] + d1[i, :] + eps16
                    a = ex_b[i, :] / den
                    ms = mf[lane]
                    for h in range(_H):
                        ahv = jnp.full((16,), a[h] * ms, jnp.float32)
                        for j in range(_F // 16):
                            s = h * _F + j * 16
                            fs_b[i, pl.ds(s, 16)] = fs_b[i, pl.ds(s, 16)] * ahv
                return cc
            lax.fori_loop(0, _C // 16, per_group, 0)
            pltpu.sync_copy(fs_b, acc.at[lidx], add=True)
            return c
        lax.fori_loop(0, nchunks, chunk, 0)
        plsc.subcore_barrier()
        pltpu.sync_copy(acc.at[pl.ds(sid * _RT, _RT)],
                        outp_hbm.at[pl.ds(lo + sid * _RT, _RT)])
        plsc.subcore_barrier()
        return carry
    lax.fori_loop(0, 5, one_pass, 0)


def _msg(fs, ex, denp, src, dst):
    mesh = plsc.VectorSubcoreMesh(core_axis_name="c", subcore_axis_name="s")
    k = pl.kernel(
        _msg_body,
        compiler_params=pltpu.CompilerParams(use_tc_tiling_on_sc=False),
        out_type=jax.ShapeDtypeStruct((_N2, _HF), jnp.float32),
        mesh=mesh,
        scratch_types=[
            pltpu.VMEM((_C,), jnp.int32),
            pltpu.VMEM((_C,), jnp.int32),
            pltpu.VMEM((_C,), jnp.int32),
            pltpu.VMEM((_C,), jnp.int32),
            pltpu.VMEM((_C, 16), jnp.float32),
            pltpu.VMEM((_C, 16), jnp.float32),
            pltpu.VMEM((_C, 16), jnp.float32),
            pltpu.VMEM((_C, 16), jnp.float32),
            pltpu.VMEM((_C, _HF), jnp.float32),
            pltpu.VMEM((40, _HF), jnp.float32),
            pltpu.VMEM_SHARED((_R, _HF), jnp.float32),
            pltpu.SemaphoreType.DMA,
        ],
    )
    return k(fs, ex, denp, src, dst)


def _relu_body(x_ref, o_ref):
    o_ref[...] = jnp.maximum(x_ref[...], 0.0)


def _relu(outp):
    return pl.pallas_call(
        _relu_body,
        grid=(_NB,),
        in_specs=[pl.BlockSpec((_BN, _HF), lambda i: (i, 0))],
        out_specs=pl.BlockSpec((_BN, _HF), lambda i: (i, 0)),
        out_shape=jax.ShapeDtypeStruct((_N, _HF), jnp.float32),
    )(outp)


@jax.jit
def kernel(feat_src, feat_dst, edge_index, dst_node_transformation_weight,
           src_node_transformation_weight, relation_embedding,
           relation_transformation_weight):
    src = edge_index[0]
    dst = edge_index[1]
    fs, es, ed = _dense(feat_src, feat_dst,
                        src_node_transformation_weight,
                        dst_node_transformation_weight,
                        relation_embedding.reshape(1, _D),
                        relation_transformation_weight)
    ex, denp = _attn(es, ed, src, dst)
    outp = _msg(fs, ex, denp, src, dst)
    return _relu(outp)


# speedup vs baseline: 5.9040x; 1.6898x over previous
import functools
import jax
import jax.numpy as jnp
from jax import lax
from jax.experimental import pallas as pl
from jax.experimental.pallas import tpu as pltpu
from jax.experimental.pallas import tpu_sc as plsc

_N = 10000
_E = 160000
_D = 256
_H = 8
_F = 64
_HF = _H * _F
_NB = 10
_BN = _N // _NB
_C = 128
_NCHUNK = _E // _C
_N2 = 10240
_R = 512
_RT = _R // 16
_CAP = 10144


def _dense_body(fsrc, fdst, wsrc, wdst, rele, wrel, fs_o, es_o, ed_o):
    f32 = jnp.float32
    rel_att = lax.dot_general(rele[...], wrel[...], (((1,), (0,)), ((), ())),
                              preferred_element_type=f32)
    qd = jnp.concatenate([rel_att[:, h * 2 * _F: h * 2 * _F + _F]
                          for h in range(_H)], axis=1)
    qs = jnp.concatenate([rel_att[:, h * 2 * _F + _F: (h + 1) * 2 * _F]
                          for h in range(_H)], axis=1)
    j64 = lax.broadcasted_iota(jnp.int32, (_HF, 16), 0) // _F
    l8 = lax.broadcasted_iota(jnp.int32, (_HF, 16), 1) % _H
    sel = jnp.where(j64 == l8, f32(1.0), f32(0.0))
    ws8 = lax.dot_general(wsrc[...] * qs, sel, (((1,), (0,)), ((), ())),
                          preferred_element_type=f32)
    wd8 = lax.dot_general(wdst[...] * qd, sel, (((1,), (0,)), ((), ())),
                          preferred_element_type=f32)
    fs_o[...] = lax.dot_general(fsrc[...], wsrc[...], (((1,), (0,)), ((), ())),
                                preferred_element_type=f32)
    es_o[...] = lax.dot_general(fsrc[...], ws8, (((1,), (0,)), ((), ())),
                                preferred_element_type=f32)
    ed_o[...] = lax.dot_general(fdst[...], wd8, (((1,), (0,)), ((), ())),
                                preferred_element_type=f32)


def _dense(feat_src, feat_dst, wsrc, wdst, rele, wrel):
    full = lambda shape: pl.BlockSpec(shape, lambda i: (0, 0))
    row = lambda w: pl.BlockSpec((_BN, w), lambda i: (i, 0))
    return pl.pallas_call(
        _dense_body,
        grid=(_NB,),
        in_specs=[row(_D), row(_D), full((_D, _HF)), full((_D, _HF)),
                  full((1, _D)), full((_D, 2 * _HF))],
        out_specs=[row(_HF), row(16), row(16)],
        out_shape=[jax.ShapeDtypeStruct((_N, _HF), jnp.float32),
                   jax.ShapeDtypeStruct((_N, 16), jnp.float32),
                   jax.ShapeDtypeStruct((_N, 16), jnp.float32)],
    )(feat_src, feat_dst, wsrc, wdst, rele, wrel)


def _attn_body(es_hbm, ed_hbm, src_hbm, dst_hbm, ex_hbm, denp_hbm,
               idx_s, idx_d, es_b, ed_b, ex_b, zb, den_sh, sem):
    cid = lax.axis_index("c")
    sid = lax.axis_index("s")
    w = cid * 16 + sid

    def zrow(i, c):
        zb[i, :] = jnp.zeros((16,), jnp.float32)
        return c
    lax.fori_loop(0, _N2 // 16, zrow, 0)
    pltpu.sync_copy(zb, den_sh.at[pl.ds(sid * (_N2 // 16), _N2 // 16)])
    plsc.subcore_barrier()

    nchunks = jnp.where(w < _NCHUNK - 32 * (_NCHUNK // 32),
                        _NCHUNK // 32 + 1, _NCHUNK // 32)

    def chunk(k, c):
        base = (w + 32 * k) * _C
        pltpu.sync_copy(src_hbm.at[pl.ds(base, _C)], idx_s)
        pltpu.sync_copy(dst_hbm.at[pl.ds(base, _C)], idx_d)
        pltpu.async_copy(es_hbm.at[idx_s], es_b, sem).wait()
        pltpu.async_copy(ed_hbm.at[idx_d], ed_b, sem).wait()

        z16 = jnp.zeros((16,), jnp.float32)
        s16 = jnp.full((16,), 0.2, jnp.float32)

        def per_edge(i, cc):
            v = es_b[i, :] + ed_b[i, :]
            v = jnp.maximum(v, z16) + s16 * jnp.minimum(v, z16)
            ex_b[i, :] = jnp.exp(v)
            return cc
        lax.fori_loop(0, _C, per_edge, 0)
        pltpu.sync_copy(ex_b, ex_hbm.at[pl.ds(base, _C)])
        pltpu.sync_copy(ex_b, den_sh.at[idx_d], add=True)
        return c
    lax.fori_loop(0, nchunks, chunk, 0)
    plsc.subcore_barrier()
    rows = _N2 // 16
    pltpu.sync_copy(den_sh.at[pl.ds(sid * rows, rows)],
                    denp_hbm.at[pl.ds(cid * _N2 + sid * rows, rows)])


def _attn(es, ed, src, dst):
    mesh = plsc.VectorSubcoreMesh(core_axis_name="c", subcore_axis_name="s")
    k = pl.kernel(
        _attn_body,
        compiler_params=pltpu.CompilerParams(use_tc_tiling_on_sc=False),
        out_type=[jax.ShapeDtypeStruct((_E, 16), jnp.float32),
                  jax.ShapeDtypeStruct((2 * _N2, 16), jnp.float32)],
        mesh=mesh,
        scratch_types=[
            pltpu.VMEM((_C,), jnp.int32),
            pltpu.VMEM((_C,), jnp.int32),
            pltpu.VMEM((_C, 16), jnp.float32),
            pltpu.VMEM((_C, 16), jnp.float32),
            pltpu.VMEM((_C, 16), jnp.float32),
            pltpu.VMEM((_N2 // 16, 16), jnp.float32),
            pltpu.VMEM_SHARED((_N2, 16), jnp.float32),
            pltpu.SemaphoreType.DMA,
        ],
    )
    return k(es, ed, src, dst)


def _msg_body(fs_hbm, ex_hbm, denp_hbm, src_hbm, dst_hbm, outp_hbm,
              idx_s, idx_d, lidx, gidx, gidx2, ex_g, d0, d1, fs_b, zb,
              csrc, cdst, ceid, acc, sem):
    cid = lax.axis_index("c")
    sid = lax.axis_index("s")
    z16i = jnp.zeros((16,), jnp.int32)
    z16f = jnp.zeros((16,), jnp.float32)
    one16 = jnp.ones((16,), jnp.float32)
    eps16 = jnp.full((16,), 1e-16, jnp.float32)
    n2v = jnp.full((16,), _N2, jnp.int32)
    m1v = jnp.full((16,), -1, jnp.int32)
    one16i = jnp.ones((16,), jnp.int32)
    iota16 = lax.iota(jnp.int32, 16)
    trash16 = jnp.full((16,), _CAP - 16, jnp.int32) + iota16

    cols = [iota16 + jnp.full((16,), j * 16, jnp.int32)
            for j in range(_HF // 16)]

    def zrow(i, c):
        rowv = jnp.full((16,), i, jnp.int32)
        for j in range(_HF // 16):
            plsc.store_scatter(zb, [rowv, cols[j]], z16f)
        return c
    lax.fori_loop(0, 32, zrow, 0)

    def zcb(i, c):
        csrc[pl.ds(i * 16, 16)] = z16i
        ceid[pl.ds(i * 16, 16)] = z16i
        return c
    lax.fori_loop(0, _CAP // 16, zcb, 0)

    nchunks = jnp.where(sid < _NCHUNK - 16 * (_NCHUNK // 16),
                        _NCHUNK // 16 + 1, _NCHUNK // 16)

    def one_pass(p, carry):
        lo = pl.multiple_of(cid * (10 * _R) + p * _R, _R)
        lov = jnp.full((16,), lo, jnp.int32)
        hiv = jnp.full((16,), lo + _R, jnp.int32)
        r0 = sid * _RT
        pltpu.sync_copy(zb, acc.at[pl.ds(r0, 32)])

        def sfill(i, c):
            cdst[pl.ds(i * 16, 16)] = m1v
            return c
        lax.fori_loop(0, _CAP // 16, sfill, 0)

        def scan_chunk(k, off):
            base = (sid + 16 * k) * _C
            pltpu.sync_copy(src_hbm.at[pl.ds(base, _C)], idx_s)
            pltpu.sync_copy(dst_hbm.at[pl.ds(base, _C)], idx_d)

            def grp(g, o):
                dvec = idx_d[pl.ds(g * 16, 16)]
                svec = idx_s[pl.ds(g * 16, 16)]
                evec = jnp.full((16,), base + g * 16, jnp.int32) + iota16
                mask = (dvec >= lov) & (dvec < hiv)
                pos = plsc.cumsum(jnp.where(mask, one16i, z16i))
                widx = jnp.where(mask,
                                 jnp.full((16,), o - 1, jnp.int32) + pos,
                                 trash16)
                plsc.store_scatter(cdst, [widx], dvec)
                plsc.store_scatter(csrc, [widx], svec)
                plsc.store_scatter(ceid, [widx], evec)
                return o + pos[15]
            return lax.fori_loop(0, _C // 16, grp, off)
        nkept = lax.fori_loop(0, nchunks, scan_chunk, jnp.int32(0))
        plsc.subcore_barrier()

        nch2 = (nkept + _C - 1) // _C

        def chunk2(q, c):
            b2 = q * _C
            pltpu.async_copy(ex_hbm.at[ceid.at[pl.ds(b2, _C)]],
                             ex_g, sem).wait()
            for g in range(_C // 16):
                dvec = cdst[pl.ds(b2 + g * 16, 16)]
                dcl = jnp.maximum(dvec, z16i)
                gidx[pl.ds(g * 16, 16)] = dcl
                gidx2[pl.ds(g * 16, 16)] = dcl + n2v
            pltpu.async_copy(denp_hbm.at[gidx], d0, sem).wait()
            pltpu.async_copy(denp_hbm.at[gidx2], d1, sem).wait()
            for half in range(2):
                hb = b2 + half * (_C // 2)
                pltpu.async_copy(fs_hbm.at[csrc.at[pl.ds(hb, _C // 2)]],
                                 fs_b, sem).wait()
                for g in range(_C // 32):
                    dvec = cdst[pl.ds(hb + g * 16, 16)]
                    inb = (dvec >= lov) & (dvec < hiv)
                    lidx[pl.ds(g * 16, 16)] = jnp.where(inb, dvec - lov,
                                                        z16i)

                def grp2(g, cc):
                    dvec = cdst[pl.ds(hb + g * 16, 16)]
                    inb = (dvec >= lov) & (dvec < hiv)
                    mf = jnp.where(inb, one16, z16f)
                    for lane in range(16):
                        i = half * (_C // 2) + g * 16 + lane
                        jj = g * 16 + lane
                        iv = jnp.full((16,), i, jnp.int32)
                        jv = jnp.full((16,), jj, jnp.int32)
                        den = (plsc.load_gather(d0, [iv, iota16])
                               + plsc.load_gather(d1, [iv, iota16]) + eps16)
                        a = plsc.load_gather(ex_g, [iv, iota16]) / den
                        ms = mf[lane]
                        for h in range(_H):
                            ahv = jnp.full((16,), a[h] * ms, jnp.float32)
                            for j in range(_F // 16):
                                cidx = cols[h * (_F // 16) + j]
                                row = plsc.load_gather(fs_b, [jv, cidx])
                                plsc.store_scatter(fs_b, [jv, cidx],
                                                   row * ahv)
                    return cc
                lax.fori_loop(0, _C // 32, grp2, 0)
                pltpu.sync_copy(fs_b, acc.at[lidx], add=True)
            return c
        lax.fori_loop(0, nch2, chunk2, 0)
        plsc.subcore_barrier()
        pltpu.sync_copy(acc.at[pl.ds(sid * _RT, _RT)],
                        outp_hbm.at[pl.ds(lo + sid * _RT, _RT)])
        plsc.subcore_barrier()
        return carry
    lax.fori_loop(0, 10, one_pass, 0)


def _msg(fs, ex, denp, src, dst):
    mesh = plsc.VectorSubcoreMesh(core_axis_name="c", subcore_axis_name="s")
    k = pl.kernel(
        _msg_body,
        compiler_params=pltpu.CompilerParams(use_tc_tiling_on_sc=False,
                                             needs_layout_passes=False),
        out_type=jax.ShapeDtypeStruct((_N2, _HF), jnp.float32),
        mesh=mesh,
        scratch_types=[
            pltpu.VMEM((_C,), jnp.int32),
            pltpu.VMEM((_C,), jnp.int32),
            pltpu.VMEM((_C // 2,), jnp.int32),
            pltpu.VMEM((_C,), jnp.int32),
            pltpu.VMEM((_C,), jnp.int32),
            pltpu.VMEM((_C, 16), jnp.float32),
            pltpu.VMEM((_C, 16), jnp.float32),
            pltpu.VMEM((_C, 16), jnp.float32),
            pltpu.VMEM((_C // 2, _HF), jnp.float32),
            pltpu.VMEM((32, _HF), jnp.float32),
            pltpu.VMEM((_CAP,), jnp.int32),
            pltpu.VMEM((_CAP,), jnp.int32),
            pltpu.VMEM((_CAP,), jnp.int32),
            pltpu.VMEM_SHARED((_R, _HF), jnp.float32),
            pltpu.SemaphoreType.DMA,
        ],
    )
    return k(fs, ex, denp, src, dst)


def _relu_body(x_ref, o_ref):
    o_ref[...] = jnp.maximum(x_ref[...], 0.0)


def _relu(outp):
    return pl.pallas_call(
        _relu_body,
        grid=(_NB,),
        in_specs=[pl.BlockSpec((_BN, _HF), lambda i: (i, 0))],
        out_specs=pl.BlockSpec((_BN, _HF), lambda i: (i, 0)),
        out_shape=jax.ShapeDtypeStruct((_N, _HF), jnp.float32),
    )(outp)


@jax.jit
def kernel(feat_src, feat_dst, edge_index, dst_node_transformation_weight,
           src_node_transformation_weight, relation_embedding,
           relation_transformation_weight):
    src = edge_index[0]
    dst = edge_index[1]
    fs, es, ed = _dense(feat_src, feat_dst,
                        src_node_transformation_weight,
                        dst_node_transformation_weight,
                        relation_embedding.reshape(1, _D),
                        relation_transformation_weight)
    ex, denp = _attn(es, ed, src, dst)
    outp = _msg(fs, ex, denp, src, dst)
    return _relu(outp)
